# baseline (device time: 24092 ns/iter reference)
import jax
import jax.numpy as jnp
from jax import lax
from jax.experimental import pallas as pl
from jax.experimental.pallas import tpu as pltpu

N_DEV = 4
E_PER_DEV = 4
N_TOK = 1024
D_MODEL = 512
D_FF = 1024
N_EXP = 16
ROWS = N_TOK // N_DEV
CAP = 96


def kernel(x, router_W, route_idx, expert_W, shared_W):
    def body(x_ref, rw_ref, idx_ref, ew_hbm, sw_hbm, out_ref,
             ew_vmem, ew_bf, sw_vmem, xs_ref, idxbf_ref,
             send_ref, recv_ref, copy_sems, send_sems, recv_sems):
        my_pos = lax.axis_index("i")

        ew_copies = []
        for e in range(E_PER_DEV):
            c = pltpu.make_async_copy(ew_hbm.at[e], ew_vmem.at[e],
                                      copy_sems.at[e])
            c.start()
            ew_copies.append(c)
        sw_copy = pltpu.make_async_copy(sw_hbm, sw_vmem,
                                        copy_sems.at[E_PER_DEV])
        sw_copy.start()

        barrier_sem = pltpu.get_barrier_semaphore()
        for off in range(1, N_DEV):
            pl.semaphore_signal(
                barrier_sem, inc=1,
                device_id=((my_pos + off) % N_DEV,),
                device_id_type=pl.DeviceIdType.MESH,
            )
        pl.semaphore_wait(barrier_sem, N_DEV - 1)

        xf = x_ref[:, :]
        scores = jnp.dot(xf, rw_ref[:, :], preferred_element_type=jnp.float32)
        scores = scores - jnp.max(scores, axis=-1, keepdims=True)
        es = jnp.exp(scores)
        probs = es / jnp.sum(es, axis=-1, keepdims=True)
        idx = idx_ref[:, :]
        lanes = lax.broadcasted_iota(jnp.int32, (N_TOK, N_EXP), 1)
        sel_prob = jnp.sum(jnp.where(lanes == idx, probs, 0.0),
                           axis=-1, keepdims=True)
        xs_ref[:, :] = (xf * sel_prob).astype(jnp.bfloat16)
        idxbf_ref[:, :] = idx.astype(jnp.bfloat16)

        tri_i = lax.broadcasted_iota(jnp.int32, (ROWS, ROWS), 0)
        tri_j = lax.broadcasted_iota(jnp.int32, (ROWS, ROWS), 1)
        tri = jnp.where(tri_j <= tri_i, 1.0, 0.0).astype(jnp.bfloat16)
        cap_iota = lax.broadcasted_iota(jnp.int32, (ROWS, CAP), 1)

        def pack_matrix(rows, owner):
            idx_c = idx_ref[rows, :]
            mine = lax.div(idx_c, E_PER_DEV) == owner
            mask_bf = jnp.where(mine, 1.0, 0.0).astype(jnp.bfloat16)
            rank = jnp.dot(tri, mask_bf, preferred_element_type=jnp.float32)
            slot = rank.astype(jnp.int32) - 1
            return jnp.where((cap_iota == slot) & mine, 1.0, 0.0)

        my_rows = pl.ds(my_pos * ROWS, ROWS)
        ew_ready = [False] * E_PER_DEV

        def packed_partial(c):
            rows = pl.ds(c * ROWS, ROWS)
            m = pack_matrix(rows, my_pos).astype(jnp.bfloat16)
            xp = lax.dot_general(m, xs_ref[rows, :], (((0,), (0,)), ((), ())),
                                 preferred_element_type=jnp.float32
                                 ).astype(jnp.bfloat16)
            ip = lax.dot_general(m, idxbf_ref[rows, :],
                                 (((0,), (0,)), ((), ())),
                                 preferred_element_type=jnp.float32)
            part = jnp.zeros((CAP, D_FF), jnp.float32)
            for e in range(E_PER_DEV):
                if not ew_ready[e]:
                    ew_copies[e].wait()
                    ew_bf[e, :, :] = ew_vmem[e, :, :].astype(jnp.bfloat16)
                    ew_ready[e] = True
                e_glob = my_pos * E_PER_DEV + e
                emask = jnp.where(ip == e_glob, 1.0, 0.0).astype(jnp.bfloat16)
                part = part + jnp.dot(xp * emask, ew_bf[e, :, :],
                                      preferred_element_type=jnp.float32)
            return part, m

        rdmas = []
        for k in (1, 0, 2):
            peer = (my_pos + 1 + k) % N_DEV
            part, _ = packed_partial(peer)
            send_ref[k, :, :] = part.astype(jnp.bfloat16)
            rdma = pltpu.make_async_remote_copy(
                src_ref=send_ref.at[k],
                dst_ref=recv_ref.at[2 - k],
                send_sem=send_sems.at[k],
                recv_sem=recv_sems.at[2 - k],
                device_id=(peer,),
                device_id_type=pl.DeviceIdType.MESH,
            )
            rdma.start()
            rdmas.append(rdma)

        m_recv = [pack_matrix(my_rows, (my_pos + 1 + j) % N_DEV)
                  .astype(jnp.bfloat16) for j in range(N_DEV - 1)]
        part, m_mine = packed_partial(my_pos)
        acc = jnp.dot(m_mine.astype(jnp.float32), part,
                      preferred_element_type=jnp.float32)
        sw_copy.wait()
        acc = acc + jnp.dot(
            x_ref[my_rows, :].astype(jnp.bfloat16),
            sw_vmem[:, :].astype(jnp.bfloat16),
            preferred_element_type=jnp.float32,
        )

        for j, rdma in ((1, rdmas[0]), (2, rdmas[1]), (0, rdmas[2])):
            rdma.wait_recv()
            acc = acc + jnp.dot(m_recv[j], recv_ref[j, :, :],
                                preferred_element_type=jnp.float32)
        out_ref[:, :] = acc
        for rdma in rdmas:
            rdma.wait_send()

    return pl.pallas_call(
        body,
        out_shape=jax.ShapeDtypeStruct((ROWS, D_FF), jnp.float32),
        in_specs=[
            pl.BlockSpec(memory_space=pltpu.VMEM),
            pl.BlockSpec(memory_space=pltpu.VMEM),
            pl.BlockSpec(memory_space=pltpu.VMEM),
            pl.BlockSpec(memory_space=pl.ANY),
            pl.BlockSpec(memory_space=pl.ANY),
        ],
        out_specs=pl.BlockSpec(memory_space=pltpu.VMEM),
        scratch_shapes=[
            pltpu.VMEM((E_PER_DEV, D_MODEL, D_FF), jnp.float32),
            pltpu.VMEM((E_PER_DEV, D_MODEL, D_FF), jnp.bfloat16),
            pltpu.VMEM((D_MODEL, D_FF), jnp.float32),
            pltpu.VMEM((N_TOK, D_MODEL), jnp.bfloat16),
            pltpu.VMEM((N_TOK, 1), jnp.bfloat16),
            pltpu.VMEM((N_DEV - 1, CAP, D_FF), jnp.bfloat16),
            pltpu.VMEM((N_DEV - 1, CAP, D_FF), jnp.bfloat16),
            pltpu.SemaphoreType.DMA((E_PER_DEV + 1,)),
            pltpu.SemaphoreType.DMA((N_DEV - 1,)),
            pltpu.SemaphoreType.DMA((N_DEV - 1,)),
        ],
        compiler_params=pltpu.CompilerParams(collective_id=0),
    )(x, router_W, route_idx, expert_W, shared_W)


# device time: 23451 ns/iter; 1.0273x vs baseline; 1.0273x over previous
import jax
import jax.numpy as jnp
from jax import lax
from jax.experimental import pallas as pl
from jax.experimental.pallas import tpu as pltpu

N_DEV = 4
E_PER_DEV = 4
N_TOK = 1024
D_MODEL = 512
D_FF = 1024
N_EXP = 16
ROWS = N_TOK // N_DEV
CAP = 96


def kernel(x, router_W, route_idx, expert_W, shared_W):
    def body(x_ref, rw_ref, idx_ref, ew_hbm, sw_hbm, out_ref,
             ew_vmem, ew_bf, sw_vmem, xs_ref, idxbf_ref,
             send_ref, recv_ref, copy_sems, send_sems, recv_sems):
        my_pos = lax.axis_index("i")

        ew_copies = []
        for e in range(E_PER_DEV):
            c = pltpu.make_async_copy(ew_hbm.at[e], ew_vmem.at[e],
                                      copy_sems.at[e])
            c.start()
            ew_copies.append(c)
        sw_copy = pltpu.make_async_copy(sw_hbm, sw_vmem,
                                        copy_sems.at[E_PER_DEV])
        sw_copy.start()

        barrier_sem = pltpu.get_barrier_semaphore()
        for off in range(1, N_DEV):
            pl.semaphore_signal(
                barrier_sem, inc=1,
                device_id=((my_pos + off) % N_DEV,),
                device_id_type=pl.DeviceIdType.MESH,
            )
        pl.semaphore_wait(barrier_sem, N_DEV - 1)

        xf = x_ref[:, :]
        scores = jnp.dot(xf, rw_ref[:, :], preferred_element_type=jnp.float32)
        scores = scores - jnp.max(scores, axis=-1, keepdims=True)
        es = jnp.exp(scores)
        probs = es / jnp.sum(es, axis=-1, keepdims=True)
        idx = idx_ref[:, :]
        lanes = lax.broadcasted_iota(jnp.int32, (N_TOK, N_EXP), 1)
        sel_prob = jnp.sum(jnp.where(lanes == idx, probs, 0.0),
                           axis=-1, keepdims=True)
        xs_ref[:, :] = (xf * sel_prob).astype(jnp.bfloat16)
        idxbf_ref[:, :] = idx.astype(jnp.bfloat16)

        tri_i = lax.broadcasted_iota(jnp.int32, (ROWS, ROWS), 0)
        tri_j = lax.broadcasted_iota(jnp.int32, (ROWS, ROWS), 1)
        tri = jnp.where(tri_j <= tri_i, 1.0, 0.0).astype(jnp.bfloat16)
        cap_iota = lax.broadcasted_iota(jnp.int32, (ROWS, CAP), 1)

        def pack_matrix(rows, owner):
            idx_c = idx_ref[rows, :]
            mine = lax.div(idx_c, E_PER_DEV) == owner
            mask_bf = jnp.where(mine, 1.0, 0.0).astype(jnp.bfloat16)
            rank = jnp.dot(tri, mask_bf, preferred_element_type=jnp.float32)
            slot = rank.astype(jnp.int32) - 1
            return jnp.where((cap_iota == slot) & mine, 1.0, 0.0)

        my_rows = pl.ds(my_pos * ROWS, ROWS)
        m_recv = [pack_matrix(my_rows, (my_pos + 1 + j) % N_DEV)
                  .astype(jnp.bfloat16) for j in range(N_DEV - 1)]

        ew_ready = [False] * E_PER_DEV

        def packed_partial(c):
            rows = pl.ds(c * ROWS, ROWS)
            m = pack_matrix(rows, my_pos).astype(jnp.bfloat16)
            xp = lax.dot_general(m, xs_ref[rows, :], (((0,), (0,)), ((), ())),
                                 preferred_element_type=jnp.float32
                                 ).astype(jnp.bfloat16)
            ip = lax.dot_general(m, idxbf_ref[rows, :],
                                 (((0,), (0,)), ((), ())),
                                 preferred_element_type=jnp.float32)
            part = jnp.zeros((CAP, D_FF), jnp.float32)
            for e in range(E_PER_DEV):
                if not ew_ready[e]:
                    ew_copies[e].wait()
                    ew_bf[e, :, :] = ew_vmem[e, :, :].astype(jnp.bfloat16)
                    ew_ready[e] = True
                e_glob = my_pos * E_PER_DEV + e
                emask = jnp.where(ip == e_glob, 1.0, 0.0).astype(jnp.bfloat16)
                part = part + jnp.dot(xp * emask, ew_bf[e, :, :],
                                      preferred_element_type=jnp.float32)
            return part, m

        rdmas = []
        for k in (1, 0, 2):
            peer = (my_pos + 1 + k) % N_DEV
            part, _ = packed_partial(peer)
            send_ref[k, :, :] = part.astype(jnp.bfloat16)
            rdma = pltpu.make_async_remote_copy(
                src_ref=send_ref.at[k],
                dst_ref=recv_ref.at[2 - k],
                send_sem=send_sems.at[k],
                recv_sem=recv_sems.at[2 - k],
                device_id=(peer,),
                device_id_type=pl.DeviceIdType.MESH,
            )
            rdma.start()
            rdmas.append(rdma)

        part, m_mine = packed_partial(my_pos)
        acc = jnp.dot(m_mine.astype(jnp.float32), part,
                      preferred_element_type=jnp.float32)
        sw_copy.wait()
        acc = acc + jnp.dot(
            x_ref[my_rows, :].astype(jnp.bfloat16),
            sw_vmem[:, :].astype(jnp.bfloat16),
            preferred_element_type=jnp.float32,
        )

        for j, rdma in ((1, rdmas[0]), (2, rdmas[1]), (0, rdmas[2])):
            rdma.wait_recv()
            acc = acc + jnp.dot(m_recv[j], recv_ref[j, :, :],
                                preferred_element_type=jnp.float32)
        out_ref[:, :] = acc
        for rdma in rdmas:
            rdma.wait_send()

    return pl.pallas_call(
        body,
        out_shape=jax.ShapeDtypeStruct((ROWS, D_FF), jnp.float32),
        in_specs=[
            pl.BlockSpec(memory_space=pltpu.VMEM),
            pl.BlockSpec(memory_space=pltpu.VMEM),
            pl.BlockSpec(memory_space=pltpu.VMEM),
            pl.BlockSpec(memory_space=pl.ANY),
            pl.BlockSpec(memory_space=pl.ANY),
        ],
        out_specs=pl.BlockSpec(memory_space=pltpu.VMEM),
        scratch_shapes=[
            pltpu.VMEM((E_PER_DEV, D_MODEL, D_FF), jnp.float32),
            pltpu.VMEM((E_PER_DEV, D_MODEL, D_FF), jnp.bfloat16),
            pltpu.VMEM((D_MODEL, D_FF), jnp.float32),
            pltpu.VMEM((N_TOK, D_MODEL), jnp.bfloat16),
            pltpu.VMEM((N_TOK, 1), jnp.bfloat16),
            pltpu.VMEM((N_DEV - 1, CAP, D_FF), jnp.bfloat16),
            pltpu.VMEM((N_DEV - 1, CAP, D_FF), jnp.bfloat16),
            pltpu.SemaphoreType.DMA((E_PER_DEV + 1,)),
            pltpu.SemaphoreType.DMA((N_DEV - 1,)),
            pltpu.SemaphoreType.DMA((N_DEV - 1,)),
        ],
        compiler_params=pltpu.CompilerParams(collective_id=0),
    )(x, router_W, route_idx, expert_W, shared_W)
